# baseline (speedup 1.0000x reference)
import functools

import jax
import jax.numpy as jnp
from jax import lax
from jax.experimental import pallas as pl
from jax.experimental.pallas import tpu as pltpu
from jax.experimental.pallas import tpu_sc as plsc

N = 100000
NP = 100352
E = 6400000
NC, NS = 2, 16
NW = NC * NS
EPW = E // NW
CH = 10000
NIT = EPW // CH
CHH = CH // 2
CH8 = 2000
NIT8 = EPW // CH8
CHH8 = CH8 // 2
RPW = NP // NS

_MESH = dict(core_axis_name="c", subcore_axis_name="s",
             num_cores=NC, num_subcores=NS)
_CP = pltpu.CompilerParams(use_tc_tiling_on_sc=False)


def _worker(c, s):
    return c * NS + s


def _sc_deg(dst, zeros_n, ones_c):
    @functools.partial(
        pl.kernel,
        out_type=jax.ShapeDtypeStruct((NC * NP,), jnp.float32),
        mesh=plsc.VectorSubcoreMesh(**_MESH),
        compiler_params=_CP,
        scratch_types=[
            pltpu.VMEM_SHARED((NP,), jnp.float32),
            pltpu.VMEM((CHH,), jnp.int32),
            pltpu.VMEM((CHH,), jnp.int32),
            pltpu.VMEM((CHH,), jnp.float32),
            pltpu.SemaphoreType.DMA,
            pltpu.SemaphoreType.DMA,
        ],
    )
    def run(dst_h, z_h, ones_h, out_h, deg_sp, ia_v, ib_v, ones_v,
            sa, sb):
        c = lax.axis_index("c")
        s = lax.axis_index("s")
        w = _worker(c, s)
        pltpu.sync_copy(z_h.at[pl.ds(s * RPW, RPW)],
                        deg_sp.at[pl.ds(s * RPW, RPW)])
        pltpu.sync_copy(ones_h, ones_v)
        plsc.subcore_barrier()

        def body(i, carry):
            base = w * EPW + i * CH
            pltpu.sync_copy(dst_h.at[pl.ds(base, CHH)], ia_v)
            da = pltpu.async_copy(ones_v, deg_sp.at[ia_v], sa, add=True)
            pltpu.sync_copy(dst_h.at[pl.ds(base + CHH, CHH)], ib_v)
            db = pltpu.async_copy(ones_v, deg_sp.at[ib_v], sb, add=True)
            da.wait()
            db.wait()
            return carry

        lax.fori_loop(0, NIT, body, 0)
        plsc.subcore_barrier()
        pltpu.sync_copy(deg_sp.at[pl.ds(s * RPW, RPW)],
                        out_h.at[pl.ds(c * NP + s * RPW, RPW)])

    return run(dst, zeros_n, ones_c)


def _sc_agg8(src, dst, y8, zeros_n8):
    @functools.partial(
        pl.kernel,
        out_type=jax.ShapeDtypeStruct((NC, NP, 8), jnp.float32),
        mesh=plsc.VectorSubcoreMesh(**_MESH),
        compiler_params=_CP,
        scratch_types=(
            [pltpu.VMEM_SHARED((NP, 8), jnp.float32)] * 2
            + [pltpu.VMEM((CHH8,), jnp.int32)] * 4
            + [pltpu.VMEM((CHH8, 8), jnp.float32)] * 2
            + [pltpu.SemaphoreType.DMA] * 4
        ),
    )
    def run(src_h, dst_h, y8_h, z_h, out_h, y_sp, agg_sp,
            sia_v, dia_v, sib_v, dib_v, ra_v, rb_v,
            sga, sgb, ssa, ssb):
        c = lax.axis_index("c")
        s = lax.axis_index("s")
        w = _worker(c, s)
        sl = pl.ds(s * RPW, RPW)
        pltpu.sync_copy(y8_h.at[sl], y_sp.at[sl])
        pltpu.sync_copy(z_h.at[sl], agg_sp.at[sl])
        plsc.subcore_barrier()

        def body(i, carry):
            base = w * EPW + i * CH8
            pltpu.sync_copy(src_h.at[pl.ds(base, CHH8)], sia_v)
            ga = pltpu.async_copy(y_sp.at[sia_v], ra_v, sga)
            pltpu.sync_copy(dst_h.at[pl.ds(base, CHH8)], dia_v)
            pltpu.sync_copy(src_h.at[pl.ds(base + CHH8, CHH8)], sib_v)
            gb = pltpu.async_copy(y_sp.at[sib_v], rb_v, sgb)
            ga.wait()
            sa = pltpu.async_copy(ra_v, agg_sp.at[dia_v], ssa, add=True)
            pltpu.sync_copy(dst_h.at[pl.ds(base + CHH8, CHH8)], dib_v)
            gb.wait()
            sb = pltpu.async_copy(rb_v, agg_sp.at[dib_v], ssb, add=True)
            sa.wait()
            sb.wait()
            return carry

        lax.fori_loop(0, NIT8, body, 0)
        plsc.subcore_barrier()
        pltpu.sync_copy(agg_sp.at[sl], out_h.at[c, sl])

    return run(src, dst, y8, zeros_n8)


def _sc_agg1(src, dst, q, zeros_n):
    @functools.partial(
        pl.kernel,
        out_type=jax.ShapeDtypeStruct((NC * NP,), jnp.float32),
        mesh=plsc.VectorSubcoreMesh(**_MESH),
        compiler_params=_CP,
        scratch_types=[
            pltpu.VMEM_SHARED((NP,), jnp.float32),
            pltpu.VMEM_SHARED((NP,), jnp.float32),
            pltpu.VMEM((CHH,), jnp.int32),
            pltpu.VMEM((CHH,), jnp.int32),
            pltpu.VMEM((CHH,), jnp.int32),
            pltpu.VMEM((CHH,), jnp.int32),
            pltpu.VMEM((CHH,), jnp.float32),
            pltpu.VMEM((CHH,), jnp.float32),
            pltpu.SemaphoreType.DMA,
            pltpu.SemaphoreType.DMA,
            pltpu.SemaphoreType.DMA,
            pltpu.SemaphoreType.DMA,
        ],
    )
    def run(src_h, dst_h, q_h, z_h, out_h, q_sp, agg_sp,
            sia_v, dia_v, sib_v, dib_v, va_v, vb_v,
            sga, sgb, ssa, ssb):
        c = lax.axis_index("c")
        s = lax.axis_index("s")
        w = _worker(c, s)
        sl = pl.ds(s * RPW, RPW)
        pltpu.sync_copy(q_h.at[sl], q_sp.at[sl])
        pltpu.sync_copy(z_h.at[sl], agg_sp.at[sl])
        plsc.subcore_barrier()

        def body(i, carry):
            base = w * EPW + i * CH
            pltpu.sync_copy(src_h.at[pl.ds(base, CHH)], sia_v)
            ga = pltpu.async_copy(q_sp.at[sia_v], va_v, sga)
            pltpu.sync_copy(dst_h.at[pl.ds(base, CHH)], dia_v)
            pltpu.sync_copy(src_h.at[pl.ds(base + CHH, CHH)], sib_v)
            gb = pltpu.async_copy(q_sp.at[sib_v], vb_v, sgb)
            ga.wait()
            sa = pltpu.async_copy(va_v, agg_sp.at[dia_v], ssa, add=True)
            pltpu.sync_copy(dst_h.at[pl.ds(base + CHH, CHH)], dib_v)
            gb.wait()
            sb = pltpu.async_copy(vb_v, agg_sp.at[dib_v], ssb, add=True)
            sa.wait()
            sb.wait()
            return carry

        lax.fori_loop(0, NIT, body, 0)
        plsc.subcore_barrier()
        pltpu.sync_copy(agg_sp.at[sl],
                        out_h.at[pl.ds(c * NP + s * RPW, RPW)])

    return run(src, dst, q, zeros_n)


_TB = 6272
_TG = NP // _TB


def _col_spec():
    return pl.BlockSpec((1, _TB), lambda i: (0, i))


def _full_spec(shape):
    return pl.BlockSpec(shape, lambda i: tuple(0 for _ in shape))


def _tc_prep(d0, d1, x0, x1, x2):
    def body(d0_r, d1_r, x0_r, x1_r, x2_r, y0_o, y1_o, y2_o, dis_o):
        deg = d0_r[...] + d1_r[...] + 1.0
        dis = lax.rsqrt(deg)
        dis_o[...] = dis
        y0_o[...] = x0_r[...] * dis
        y1_o[...] = x1_r[...] * dis
        y2_o[...] = x2_r[...] * dis

    return pl.pallas_call(
        body,
        grid=(_TG,),
        in_specs=[_col_spec()] * 5,
        out_specs=[_col_spec()] * 4,
        out_shape=[jax.ShapeDtypeStruct((1, NP), jnp.float32)] * 4,
    )(d0, d1, x0, x1, x2)


def _tc_mid(a00, a01, a10, a11, a20, a21, y0, y1, y2, dis, w1t, b1c, w2t):
    def body(a00_r, a01_r, a10_r, a11_r, a20_r, a21_r,
             y0_r, y1_r, y2_r, dis_r, w1_r, b1_r, w2_r, qs_o):
        z0 = (a00_r[...] + a01_r[...] + y0_r[...]) * dis_r[...]
        z1 = (a10_r[...] + a11_r[...] + y1_r[...]) * dis_r[...]
        z2 = (a20_r[...] + a21_r[...] + y2_r[...]) * dis_r[...]
        z = jnp.concatenate([z0, z1, z2], axis=0)
        h = jnp.dot(w1_r[...], z, preferred_element_type=jnp.float32)
        h = jnp.maximum(h + b1_r[...], 0.0)
        q = jnp.dot(w2_r[...], h, preferred_element_type=jnp.float32)
        qs_o[...] = q * dis_r[...]

    return pl.pallas_call(
        body,
        grid=(_TG,),
        in_specs=[_col_spec()] * 10 + [_full_spec((16, 3)),
                                       _full_spec((16, 1)),
                                       _full_spec((1, 16))],
        out_specs=_col_spec(),
        out_shape=jax.ShapeDtypeStruct((1, NP), jnp.float32),
    )(a00, a01, a10, a11, a20, a21, y0, y1, y2, dis, w1t, b1c, w2t)


def _tc_final(aq0, aq1, qs, dis, b2r):
    def body(aq0_r, aq1_r, qs_r, dis_r, b2_r, o_r):
        o_r[...] = (aq0_r[...] + aq1_r[...] + qs_r[...]) * dis_r[...] + b2_r[...]

    return pl.pallas_call(
        body,
        grid=(_TG,),
        in_specs=[_col_spec()] * 4 + [_full_spec((1, 1))],
        out_specs=_col_spec(),
        out_shape=jax.ShapeDtypeStruct((1, NP), jnp.float32),
    )(aq0, aq1, qs, dis, b2r)


def kernel(x, edge_index, W1, b1, W2, b2):
    src = edge_index[0]
    dst = edge_index[1]

    xp = jnp.zeros((NP, 3), jnp.float32).at[:N].set(x)
    x0 = xp[:, 0].reshape(1, NP)
    x1 = xp[:, 1].reshape(1, NP)
    x2 = xp[:, 2].reshape(1, NP)
    w1t = W1.T
    b1c = b1.reshape(16, 1)
    w2t = W2.T
    b2r = b2.reshape(1, 1)
    zeros_n = jnp.zeros((NP,), jnp.float32)
    ones_c = jnp.ones((CHH,), jnp.float32)

    degp = _sc_deg(dst, zeros_n, ones_c)
    y0, y1, y2, dis = _tc_prep(degp[:NP].reshape(1, NP),
                               degp[NP:].reshape(1, NP), x0, x1, x2)
    y8 = jnp.concatenate(
        [y0, y1, y2, jnp.zeros((5, NP), jnp.float32)], axis=0).T
    aggp = _sc_agg8(src, dst, y8, jnp.zeros((NP, 8), jnp.float32))
    qs = _tc_mid(aggp[0, :, 0].reshape(1, NP), aggp[1, :, 0].reshape(1, NP),
                 aggp[0, :, 1].reshape(1, NP), aggp[1, :, 1].reshape(1, NP),
                 aggp[0, :, 2].reshape(1, NP), aggp[1, :, 2].reshape(1, NP),
                 y0, y1, y2, dis, w1t, b1c, w2t)
    aggqp = _sc_agg1(src, dst, qs.reshape(NP), zeros_n)
    out = _tc_final(aggqp[:NP].reshape(1, NP), aggqp[NP:].reshape(1, NP),
                    qs, dis, b2r)
    return out.reshape(NP, 1)[:N]


# speedup vs baseline: 1.1438x; 1.1438x over previous
import functools

import jax
import jax.numpy as jnp
from jax import lax
from jax.experimental import pallas as pl
from jax.experimental.pallas import tpu as pltpu
from jax.experimental.pallas import tpu_sc as plsc

N = 100000
NP = 100352
E = 6400000
NC, NS = 2, 16
NW = NC * NS
EPW = E // NW
CH = 10000
NIT = EPW // CH
CHH = CH // 2
CH8 = 2000
NIT8 = EPW // CH8
CHH8 = CH8 // 2
RPW = NP // NS

_MESH = dict(core_axis_name="c", subcore_axis_name="s",
             num_cores=NC, num_subcores=NS)
_CP = pltpu.CompilerParams(use_tc_tiling_on_sc=False)


def _worker(c, s):
    return c * NS + s


def _sc_deg(dst, zeros_n, ones_c):
    @functools.partial(
        pl.kernel,
        out_type=jax.ShapeDtypeStruct((NC * NP,), jnp.float32),
        mesh=plsc.VectorSubcoreMesh(**_MESH),
        compiler_params=_CP,
        scratch_types=[
            pltpu.VMEM_SHARED((NP,), jnp.float32),
            pltpu.VMEM((CHH,), jnp.int32),
            pltpu.VMEM((CHH,), jnp.int32),
            pltpu.VMEM((CHH,), jnp.float32),
            pltpu.SemaphoreType.DMA,
            pltpu.SemaphoreType.DMA,
        ],
    )
    def run(dst_h, z_h, ones_h, out_h, deg_sp, ia_v, ib_v, ones_v,
            sa, sb):
        c = lax.axis_index("c")
        s = lax.axis_index("s")
        w = _worker(c, s)
        pltpu.sync_copy(z_h.at[pl.ds(s * RPW, RPW)],
                        deg_sp.at[pl.ds(s * RPW, RPW)])
        pltpu.sync_copy(ones_h, ones_v)
        plsc.subcore_barrier()

        def body(i, carry):
            base = w * EPW + i * CH
            pltpu.sync_copy(dst_h.at[pl.ds(base, CHH)], ia_v)
            da = pltpu.async_copy(ones_v, deg_sp.at[ia_v], sa, add=True)
            pltpu.sync_copy(dst_h.at[pl.ds(base + CHH, CHH)], ib_v)
            db = pltpu.async_copy(ones_v, deg_sp.at[ib_v], sb, add=True)
            da.wait()
            db.wait()
            return carry

        lax.fori_loop(0, NIT, body, 0)
        plsc.subcore_barrier()
        pltpu.sync_copy(deg_sp.at[pl.ds(s * RPW, RPW)],
                        out_h.at[pl.ds(c * NP + s * RPW, RPW)])

    return run(dst, zeros_n, ones_c)


def _sc_agg3(src, dst, y0, y1, y2, zeros_n):
    @functools.partial(
        pl.kernel,
        out_type=[jax.ShapeDtypeStruct((NC * NP,), jnp.float32)] * 3,
        mesh=plsc.VectorSubcoreMesh(**_MESH),
        compiler_params=_CP,
        scratch_types=(
            [pltpu.VMEM_SHARED((NP,), jnp.float32)] * 6
            + [pltpu.VMEM((CH,), jnp.int32)] * 2
            + [pltpu.VMEM((CH,), jnp.float32)] * 3
            + [pltpu.SemaphoreType.DMA] * 6
        ),
    )
    def run(src_h, dst_h, y0_h, y1_h, y2_h, z_h, o0_h, o1_h, o2_h,
            y0_sp, y1_sp, y2_sp, a0_sp, a1_sp, a2_sp,
            si_v, di_v, v0_v, v1_v, v2_v,
            sg0, sg1, sg2, ss0, ss1, ss2):
        c = lax.axis_index("c")
        s = lax.axis_index("s")
        w = _worker(c, s)
        sl = pl.ds(s * RPW, RPW)
        pltpu.sync_copy(y0_h.at[sl], y0_sp.at[sl])
        pltpu.sync_copy(y1_h.at[sl], y1_sp.at[sl])
        pltpu.sync_copy(y2_h.at[sl], y2_sp.at[sl])
        pltpu.sync_copy(z_h.at[sl], a0_sp.at[sl])
        pltpu.sync_copy(z_h.at[sl], a1_sp.at[sl])
        pltpu.sync_copy(z_h.at[sl], a2_sp.at[sl])
        plsc.subcore_barrier()

        def body(i, carry):
            base = w * EPW + i * CH
            pltpu.sync_copy(src_h.at[pl.ds(base, CH)], si_v)
            pltpu.sync_copy(dst_h.at[pl.ds(base, CH)], di_v)
            g0 = pltpu.async_copy(y0_sp.at[si_v], v0_v, sg0)
            g1 = pltpu.async_copy(y1_sp.at[si_v], v1_v, sg1)
            g2 = pltpu.async_copy(y2_sp.at[si_v], v2_v, sg2)
            g0.wait()
            s0 = pltpu.async_copy(v0_v, a0_sp.at[di_v], ss0, add=True)
            g1.wait()
            s1 = pltpu.async_copy(v1_v, a1_sp.at[di_v], ss1, add=True)
            g2.wait()
            s2 = pltpu.async_copy(v2_v, a2_sp.at[di_v], ss2, add=True)
            s0.wait()
            s1.wait()
            s2.wait()
            return carry

        lax.fori_loop(0, NIT, body, 0)
        plsc.subcore_barrier()
        osl = pl.ds(c * NP + s * RPW, RPW)
        pltpu.sync_copy(a0_sp.at[sl], o0_h.at[osl])
        pltpu.sync_copy(a1_sp.at[sl], o1_h.at[osl])
        pltpu.sync_copy(a2_sp.at[sl], o2_h.at[osl])

    return run(src, dst, y0, y1, y2, zeros_n)


def _sc_agg1(src, dst, q, zeros_n):
    @functools.partial(
        pl.kernel,
        out_type=jax.ShapeDtypeStruct((NC * NP,), jnp.float32),
        mesh=plsc.VectorSubcoreMesh(**_MESH),
        compiler_params=_CP,
        scratch_types=[
            pltpu.VMEM_SHARED((NP,), jnp.float32),
            pltpu.VMEM_SHARED((NP,), jnp.float32),
            pltpu.VMEM((CHH,), jnp.int32),
            pltpu.VMEM((CHH,), jnp.int32),
            pltpu.VMEM((CHH,), jnp.int32),
            pltpu.VMEM((CHH,), jnp.int32),
            pltpu.VMEM((CHH,), jnp.float32),
            pltpu.VMEM((CHH,), jnp.float32),
            pltpu.SemaphoreType.DMA,
            pltpu.SemaphoreType.DMA,
            pltpu.SemaphoreType.DMA,
            pltpu.SemaphoreType.DMA,
        ],
    )
    def run(src_h, dst_h, q_h, z_h, out_h, q_sp, agg_sp,
            sia_v, dia_v, sib_v, dib_v, va_v, vb_v,
            sga, sgb, ssa, ssb):
        c = lax.axis_index("c")
        s = lax.axis_index("s")
        w = _worker(c, s)
        sl = pl.ds(s * RPW, RPW)
        pltpu.sync_copy(q_h.at[sl], q_sp.at[sl])
        pltpu.sync_copy(z_h.at[sl], agg_sp.at[sl])
        plsc.subcore_barrier()

        def body(i, carry):
            base = w * EPW + i * CH
            pltpu.sync_copy(src_h.at[pl.ds(base, CHH)], sia_v)
            ga = pltpu.async_copy(q_sp.at[sia_v], va_v, sga)
            pltpu.sync_copy(dst_h.at[pl.ds(base, CHH)], dia_v)
            pltpu.sync_copy(src_h.at[pl.ds(base + CHH, CHH)], sib_v)
            gb = pltpu.async_copy(q_sp.at[sib_v], vb_v, sgb)
            ga.wait()
            sa = pltpu.async_copy(va_v, agg_sp.at[dia_v], ssa, add=True)
            pltpu.sync_copy(dst_h.at[pl.ds(base + CHH, CHH)], dib_v)
            gb.wait()
            sb = pltpu.async_copy(vb_v, agg_sp.at[dib_v], ssb, add=True)
            sa.wait()
            sb.wait()
            return carry

        lax.fori_loop(0, NIT, body, 0)
        plsc.subcore_barrier()
        pltpu.sync_copy(agg_sp.at[sl],
                        out_h.at[pl.ds(c * NP + s * RPW, RPW)])

    return run(src, dst, q, zeros_n)


_TB = 6272
_TG = NP // _TB


def _col_spec():
    return pl.BlockSpec((1, _TB), lambda i: (0, i))


def _full_spec(shape):
    return pl.BlockSpec(shape, lambda i: tuple(0 for _ in shape))


def _tc_prep(d0, d1, x0, x1, x2):
    def body(d0_r, d1_r, x0_r, x1_r, x2_r, y0_o, y1_o, y2_o, dis_o):
        deg = d0_r[...] + d1_r[...] + 1.0
        dis = lax.rsqrt(deg)
        dis_o[...] = dis
        y0_o[...] = x0_r[...] * dis
        y1_o[...] = x1_r[...] * dis
        y2_o[...] = x2_r[...] * dis

    return pl.pallas_call(
        body,
        grid=(_TG,),
        in_specs=[_col_spec()] * 5,
        out_specs=[_col_spec()] * 4,
        out_shape=[jax.ShapeDtypeStruct((1, NP), jnp.float32)] * 4,
    )(d0, d1, x0, x1, x2)


def _tc_mid(a00, a01, a10, a11, a20, a21, y0, y1, y2, dis, w1t, b1c, w2t):
    def body(a00_r, a01_r, a10_r, a11_r, a20_r, a21_r,
             y0_r, y1_r, y2_r, dis_r, w1_r, b1_r, w2_r, qs_o):
        z0 = (a00_r[...] + a01_r[...] + y0_r[...]) * dis_r[...]
        z1 = (a10_r[...] + a11_r[...] + y1_r[...]) * dis_r[...]
        z2 = (a20_r[...] + a21_r[...] + y2_r[...]) * dis_r[...]
        z = jnp.concatenate([z0, z1, z2], axis=0)
        h = jnp.dot(w1_r[...], z, preferred_element_type=jnp.float32)
        h = jnp.maximum(h + b1_r[...], 0.0)
        q = jnp.dot(w2_r[...], h, preferred_element_type=jnp.float32)
        qs_o[...] = q * dis_r[...]

    return pl.pallas_call(
        body,
        grid=(_TG,),
        in_specs=[_col_spec()] * 10 + [_full_spec((16, 3)),
                                       _full_spec((16, 1)),
                                       _full_spec((1, 16))],
        out_specs=_col_spec(),
        out_shape=jax.ShapeDtypeStruct((1, NP), jnp.float32),
    )(a00, a01, a10, a11, a20, a21, y0, y1, y2, dis, w1t, b1c, w2t)


def _tc_final(aq0, aq1, qs, dis, b2r):
    def body(aq0_r, aq1_r, qs_r, dis_r, b2_r, o_r):
        o_r[...] = (aq0_r[...] + aq1_r[...] + qs_r[...]) * dis_r[...] + b2_r[...]

    return pl.pallas_call(
        body,
        grid=(_TG,),
        in_specs=[_col_spec()] * 4 + [_full_spec((1, 1))],
        out_specs=_col_spec(),
        out_shape=jax.ShapeDtypeStruct((1, NP), jnp.float32),
    )(aq0, aq1, qs, dis, b2r)


def kernel(x, edge_index, W1, b1, W2, b2):
    src = edge_index[0]
    dst = edge_index[1]

    xp = jnp.zeros((NP, 3), jnp.float32).at[:N].set(x)
    x0 = xp[:, 0].reshape(1, NP)
    x1 = xp[:, 1].reshape(1, NP)
    x2 = xp[:, 2].reshape(1, NP)
    w1t = W1.T
    b1c = b1.reshape(16, 1)
    w2t = W2.T
    b2r = b2.reshape(1, 1)
    zeros_n = jnp.zeros((NP,), jnp.float32)
    ones_c = jnp.ones((CHH,), jnp.float32)

    degp = _sc_deg(dst, zeros_n, ones_c)
    y0, y1, y2, dis = _tc_prep(degp[:NP].reshape(1, NP),
                               degp[NP:].reshape(1, NP), x0, x1, x2)
    a0, a1, a2 = _sc_agg3(src, dst, y0.reshape(NP), y1.reshape(NP),
                          y2.reshape(NP), zeros_n)
    qs = _tc_mid(a0[:NP].reshape(1, NP), a0[NP:].reshape(1, NP),
                 a1[:NP].reshape(1, NP), a1[NP:].reshape(1, NP),
                 a2[:NP].reshape(1, NP), a2[NP:].reshape(1, NP),
                 y0, y1, y2, dis, w1t, b1c, w2t)
    aggqp = _sc_agg1(src, dst, qs.reshape(NP), zeros_n)
    out = _tc_final(aggqp[:NP].reshape(1, NP), aggqp[NP:].reshape(1, NP),
                    qs, dis, b2r)
    return out.reshape(NP, 1)[:N]
